# baseline (speedup 1.0000x reference)
import functools

import jax
import jax.numpy as jnp
import numpy as np
from jax import lax
from jax.experimental import pallas as pl
from jax.experimental.pallas import tpu as pltpu
from jax.experimental.pallas import tpu_sc as plsc

NC, NS, L = 2, 16, 16
NW = NC * NS
N = 1_000_000
F = 26
GW = L * F
BASE_G = 1953
BASE_ROWS = BASE_G * L
CH_G = 31
CH_ROWS = CH_G * L
N_CH = BASE_G // CH_G
FBLOCKS = ((0, 13), (13, 13))
BIG = np.float32(1e30)


def _tile_body(x_hbm, par_hbm, out_hbm, xb, ob, parb, accb):
    cid = lax.axis_index("c")
    sid = lax.axis_index("s")
    w = sid * NC + cid
    row0 = w * BASE_ROWS + L * jnp.minimum(w, 4)

    pltpu.sync_copy(par_hbm, parb)

    iot = lax.iota(jnp.int32, L)

    def _param(base, f):
        vec = parb[pl.ds(base + 16 * (f // 16), 16)]
        return jnp.broadcast_to(vec[f % 16], (L,))

    def process(n_groups):
        for bi, (fstart, nf) in enumerate(FBLOCKS):
            last = bi == len(FBLOCKS) - 1
            pv = [_param(0, fstart + j) for j in range(nf)]
            qv = [_param(32, fstart + j) for j in range(nf)]
            gv = [_param(64, fstart + j) for j in range(nf)]

            colv = [iot * 0 + (fstart + j) for j in range(nf)]

            def gbody(g, _, pv=pv, qv=qv, gv=gv, colv=colv, bi=bi, last=last,
                      nf=nf):
                rowv = iot + g * L
                if bi == 0:
                    acc = jnp.full((L,), BIG, dtype=jnp.float32)
                else:
                    acc = accb[pl.ds(g * L, L)]
                for j in range(nf):
                    xv = plsc.bitcast(
                        plsc.load_gather(xb, [rowv, colv[j]]), jnp.float32
                    )
                    u = pv[j] - xv
                    v = xv - qv[j]
                    t = jnp.minimum(u, v) * gv[j]
                    acc = jnp.minimum(acc, t)
                if last:
                    y = jnp.where(acc >= 0.0, np.float32(1.0), np.float32(0.0))
                    ob[pl.ds(g * L, L)] = y
                else:
                    accb[pl.ds(g * L, L)] = acc
                return 0

            lax.fori_loop(0, n_groups, gbody, 0)

    def chunk_body(ci, _):
        r = row0 + ci * CH_ROWS
        pltpu.sync_copy(x_hbm.at[pl.ds(r, CH_ROWS)], xb)
        process(CH_G)
        pltpu.sync_copy(ob, out_hbm.at[pl.ds(r, CH_ROWS)])
        return 0

    lax.fori_loop(0, N_CH, chunk_body, 0)

    @pl.when(w < 4)
    def _extra():
        r = row0 + BASE_ROWS
        pltpu.sync_copy(x_hbm.at[pl.ds(r, L)], xb.at[pl.ds(0, L)])
        process(1)
        pltpu.sync_copy(ob.at[pl.ds(0, L)], out_hbm.at[pl.ds(r, L)])


_mesh = plsc.VectorSubcoreMesh(
    core_axis_name="c", subcore_axis_name="s", num_cores=NC, num_subcores=NS
)

_cc_call = functools.partial(
    pl.kernel,
    out_type=jax.ShapeDtypeStruct((N,), jnp.float32),
    mesh=_mesh,
    compiler_params=pltpu.CompilerParams(needs_layout_passes=False),
    scratch_types=[
        pltpu.VMEM((CH_ROWS, F), jnp.int32),
        pltpu.VMEM((CH_ROWS,), jnp.float32),
        pltpu.VMEM((96,), jnp.float32),
        pltpu.VMEM((CH_ROWS,), jnp.float32),
    ],
)(_tile_body)


@jax.jit
def kernel(x, cuts, cases):
    lo = cuts[:, 0]
    hi = cuts[:, 1]
    c = cases
    p = jnp.where(c == 0, lo, jnp.where(c == 1, BIG, hi))
    q = jnp.where(c == 0, -BIG, lo)
    g = jnp.where(c == 3, np.float32(-1.0), np.float32(1.0))
    params = (
        jnp.zeros((96,), jnp.float32)
        .at[0:26].set(p.astype(jnp.float32))
        .at[32:58].set(q.astype(jnp.float32))
        .at[64:90].set(g)
    )
    xi = lax.bitcast_convert_type(x, jnp.int32)
    y = _cc_call(xi, params)
    return jnp.stack([np.float32(1.0) - y, y], axis=-1)


# speedup vs baseline: 1.1209x; 1.1209x over previous
import functools

import jax
import jax.numpy as jnp
import numpy as np
from jax import lax
from jax.experimental import pallas as pl
from jax.experimental.pallas import tpu as pltpu
from jax.experimental.pallas import tpu_sc as plsc

NC, NS, L = 2, 16, 16
NW = NC * NS
N = 1_000_000
F = 26
GW = L * F
BASE_G = 1953
BASE_ROWS = BASE_G * L
CH_G = 93
CH_ROWS = CH_G * L
CH_W = CH_ROWS * F
N_CH = BASE_G // CH_G
FBLOCKS = ((0, 13), (13, 13))
BIG = np.float32(1e30)


def _tile_body(x_hbm, par_hbm, out_hbm, xb, ob, parb, accb):
    cid = lax.axis_index("c")
    sid = lax.axis_index("s")
    w = sid * NC + cid
    row0 = w * BASE_ROWS + L * jnp.minimum(w, 4)

    pltpu.sync_copy(par_hbm, parb)

    iot = lax.iota(jnp.int32, L)
    iotF = iot * F

    def _param(base, f):
        vec = parb[pl.ds(base + 16 * (f // 16), 16)]
        return jnp.broadcast_to(vec[f % 16], (L,))

    def process(n_groups):
        for bi, (fstart, nf) in enumerate(FBLOCKS):
            last = bi == len(FBLOCKS) - 1
            pv = [_param(0, fstart + j) for j in range(nf)]
            qv = [_param(32, fstart + j) for j in range(nf)]
            gv = [_param(64, fstart + j) for j in range(nf)]

            def gbody(g, _, pv=pv, qv=qv, gv=gv, bi=bi, last=last, nf=nf,
                      fstart=fstart):
                gbase = iotF + g * GW
                if bi == 0:
                    acc = jnp.full((L,), BIG, dtype=jnp.float32)
                else:
                    acc = accb[pl.ds(g * L, L)]
                for j in range(nf):
                    xv = plsc.load_gather(xb, [gbase + (fstart + j)])
                    u = pv[j] - xv
                    v = xv - qv[j]
                    t = jnp.minimum(u, v) * gv[j]
                    acc = jnp.minimum(acc, t)
                if last:
                    y = jnp.where(acc >= 0.0, np.float32(1.0), np.float32(0.0))
                    ob[pl.ds(g * L, L)] = y
                else:
                    accb[pl.ds(g * L, L)] = acc
                return 0

            lax.fori_loop(0, n_groups, gbody, 0)

    def chunk_body(ci, _):
        r = row0 + ci * CH_ROWS
        pltpu.sync_copy(x_hbm.at[pl.ds(r * F, CH_W)], xb)
        process(CH_G)
        pltpu.sync_copy(ob, out_hbm.at[pl.ds(r, CH_ROWS)])
        return 0

    lax.fori_loop(0, N_CH, chunk_body, 0)

    @pl.when(w < 4)
    def _extra():
        r = row0 + BASE_ROWS
        pltpu.sync_copy(x_hbm.at[pl.ds(r * F, GW)], xb.at[pl.ds(0, GW)])
        process(1)
        pltpu.sync_copy(ob.at[pl.ds(0, L)], out_hbm.at[pl.ds(r, L)])


_mesh = plsc.VectorSubcoreMesh(
    core_axis_name="c", subcore_axis_name="s", num_cores=NC, num_subcores=NS
)

_cc_call = functools.partial(
    pl.kernel,
    out_type=jax.ShapeDtypeStruct((N,), jnp.float32),
    mesh=_mesh,
    compiler_params=pltpu.CompilerParams(needs_layout_passes=False),
    scratch_types=[
        pltpu.VMEM((CH_W,), jnp.float32),
        pltpu.VMEM((CH_ROWS,), jnp.float32),
        pltpu.VMEM((96,), jnp.float32),
        pltpu.VMEM((CH_ROWS,), jnp.float32),
    ],
)(_tile_body)


@jax.jit
def kernel(x, cuts, cases):
    lo = cuts[:, 0]
    hi = cuts[:, 1]
    c = cases
    p = jnp.where(c == 0, lo, jnp.where(c == 1, BIG, hi))
    q = jnp.where(c == 0, -BIG, lo)
    g = jnp.where(c == 3, np.float32(-1.0), np.float32(1.0))
    params = (
        jnp.zeros((96,), jnp.float32)
        .at[0:26].set(p.astype(jnp.float32))
        .at[32:58].set(q.astype(jnp.float32))
        .at[64:90].set(g)
    )
    xlin = jnp.minimum(x.reshape(N * F), BIG)
    y = _cc_call(xlin, params)
    return jnp.stack([np.float32(1.0) - y, y], axis=-1)


# speedup vs baseline: 1.3658x; 1.2184x over previous
import functools

import jax
import jax.numpy as jnp
import numpy as np
from jax import lax
from jax.experimental import pallas as pl
from jax.experimental.pallas import tpu as pltpu
from jax.experimental.pallas import tpu_sc as plsc

NC, NS, L = 2, 16, 16
NW = NC * NS
N = 1_000_000
F = 26
BASE_G = 1953
BASE_ROWS = BASE_G * L
CH_G = 21
CH_ROWS = CH_G * L
N_CH = BASE_G // CH_G
N_PAIR = (N_CH - 1) // 2
FBLOCKS = ((0, 13), (13, 13))
BIG = np.float32(1e30)


def _tile_body(x_hbm, par_hbm, out_hbm, xb0, xb1, ob, parb, accb, sem0, sem1):
    cid = lax.axis_index("c")
    sid = lax.axis_index("s")
    w = sid * NC + cid
    row0 = w * BASE_ROWS + L * jnp.minimum(w, 4)

    pltpu.sync_copy(par_hbm, parb)

    iot = lax.iota(jnp.int32, L)

    def _param(base, f):
        vec = parb[pl.ds(base + 16 * (f // 16), 16)]
        return jnp.broadcast_to(vec[f % 16], (L,))

    def process(n_groups, xb):
        for bi, (fstart, nf) in enumerate(FBLOCKS):
            last = bi == len(FBLOCKS) - 1
            pv = [_param(0, fstart + j) for j in range(nf)]
            qv = [_param(32, fstart + j) for j in range(nf)]
            gv = [_param(64, fstart + j) for j in range(nf)]
            colv = [iot * 0 + (fstart + j) for j in range(nf)]

            def gbody(g, _, pv=pv, qv=qv, gv=gv, colv=colv, bi=bi, last=last,
                      nf=nf, xb=xb):
                rowv = iot + g * L
                if bi == 0:
                    acc = jnp.full((L,), BIG, dtype=jnp.float32)
                else:
                    acc = accb[pl.ds(g * L, L)]
                for j in range(nf):
                    xv = plsc.load_gather(xb, [rowv, colv[j]])
                    u = pv[j] - xv
                    v = xv - qv[j]
                    t = jnp.minimum(u, v) * gv[j]
                    acc = jnp.minimum(acc, t)
                if last:
                    y = jnp.where(acc >= 0.0, np.float32(1.0), np.float32(0.0))
                    ob[pl.ds(g * L, L)] = y
                else:
                    accb[pl.ds(g * L, L)] = acc
                return 0

            lax.fori_loop(0, n_groups, gbody, 0)

    def start_in(ci, xb, sem):
        r = row0 + ci * CH_ROWS
        pltpu.async_copy(x_hbm.at[pl.ds(r, CH_ROWS)], xb, sem)

    def finish(ci, xb, sem):
        r = row0 + ci * CH_ROWS
        pltpu.make_async_copy(x_hbm.at[pl.ds(r, CH_ROWS)], xb, sem).wait()
        process(CH_G, xb)
        pltpu.sync_copy(ob, out_hbm.at[pl.ds(r, CH_ROWS)])

    start_in(0, xb0, sem0)

    def pair_body(p, _):
        c0 = 2 * p
        start_in(c0 + 1, xb1, sem1)
        finish(c0, xb0, sem0)
        start_in(c0 + 2, xb0, sem0)
        finish(c0 + 1, xb1, sem1)
        return 0

    lax.fori_loop(0, N_PAIR, pair_body, 0)
    finish(N_CH - 1, xb0, sem0)

    @pl.when(w < 4)
    def _extra():
        r = row0 + BASE_ROWS
        pltpu.sync_copy(x_hbm.at[pl.ds(r, L)], xb1.at[pl.ds(0, L)])
        process(1, xb1)
        pltpu.sync_copy(ob.at[pl.ds(0, L)], out_hbm.at[pl.ds(r, L)])


_mesh = plsc.VectorSubcoreMesh(
    core_axis_name="c", subcore_axis_name="s", num_cores=NC, num_subcores=NS
)

_cc_call = functools.partial(
    pl.kernel,
    out_type=jax.ShapeDtypeStruct((N,), jnp.float32),
    mesh=_mesh,
    compiler_params=pltpu.CompilerParams(needs_layout_passes=False),
    scratch_types=[
        pltpu.VMEM((CH_ROWS, F), jnp.float32),
        pltpu.VMEM((CH_ROWS, F), jnp.float32),
        pltpu.VMEM((CH_ROWS,), jnp.float32),
        pltpu.VMEM((96,), jnp.float32),
        pltpu.VMEM((CH_ROWS,), jnp.float32),
        pltpu.SemaphoreType.DMA,
        pltpu.SemaphoreType.DMA,
    ],
)(_tile_body)


@jax.jit
def kernel(x, cuts, cases):
    lo = cuts[:, 0]
    hi = cuts[:, 1]
    c = cases
    p = jnp.where(c == 0, lo, jnp.where(c == 1, BIG, hi))
    q = jnp.where(c == 0, -BIG, lo)
    g = jnp.where(c == 3, np.float32(-1.0), np.float32(1.0))
    params = (
        jnp.zeros((96,), jnp.float32)
        .at[0:26].set(p.astype(jnp.float32))
        .at[32:58].set(q.astype(jnp.float32))
        .at[64:90].set(g)
    )
    y = _cc_call(x, params)
    return jnp.stack([np.float32(1.0) - y, y], axis=-1)
